# baseline (speedup 1.0000x reference)
import functools

import jax
import jax.numpy as jnp
from jax import lax
from jax.experimental import pallas as pl
from jax.experimental.pallas import tpu as pltpu
from jax.experimental.pallas import tpu_sc as plsc

N = 10000
E = 160000
DIN, DH, DOUT = 256, 512, 64
NC, NS = 2, 16
NW = NC * NS
EPT = E // NW
EB = 128
NB = 40
EPTP = NB * EB
CF = 128
NCH = DH // CF
ACC_ROWS = 10112
ZR = ACC_ROWS // NS
RB = 1000
GRID = N // RB

def _mesh():
    return plsc.VectorSubcoreMesh(
        core_axis_name="c", subcore_axis_name="s",
        num_cores=NC, num_subcores=NS)


def _deg_body(dst_hbm, zeros_hbm, ones_hbm, out_hbm, dst_v, ones_v, deg_sh):
    ci = lax.axis_index("c")
    si = lax.axis_index("s")
    wid = ci * NS + si
    pltpu.sync_copy(dst_hbm.at[wid], dst_v)
    pltpu.sync_copy(ones_hbm, ones_v)
    pltpu.sync_copy(zeros_hbm, deg_sh.at[pl.ds(si * ZR, ZR)])
    plsc.subcore_barrier()

    def body(j, carry):
        pltpu.sync_copy(ones_v, deg_sh.at[dst_v.at[j]], add=True)
        return carry

    lax.fori_loop(0, NB, body, 0)
    plsc.subcore_barrier()
    pltpu.sync_copy(deg_sh.at[pl.ds(si * ZR, ZR)],
                    out_hbm.at[ci].at[pl.ds(si * ZR, ZR)])


@functools.cache
def _deg():
    return pl.kernel(
        _deg_body,
        out_type=jax.ShapeDtypeStruct((NC, ACC_ROWS, CF), jnp.float32),
        mesh=_mesh(),
        scratch_types=[
            pltpu.VMEM((NB, EB), jnp.int32),
            pltpu.VMEM((EB, CF), jnp.float32),
            pltpu.VMEM_SHARED((ACC_ROWS, CF), jnp.float32),
        ],
    )


def _scat_body(y0, y1, y2, y3, src_hbm, dst_hbm, zeros_hbm,
               o0, o1, o2, o3, src_v, dst_v, rows_v, gsem, ssem, acc_sh):
    ys = (y0, y1, y2, y3)
    outs = (o0, o1, o2, o3)
    ci = lax.axis_index("c")
    si = lax.axis_index("s")
    wid = ci * NS + si
    pltpu.sync_copy(src_hbm.at[wid], src_v)
    pltpu.sync_copy(dst_hbm.at[wid], dst_v)
    for c in range(NCH):
        yc = ys[c]

        def g_issue(j, b, _y=yc):
            pltpu.async_copy(_y.at[src_v.at[j]], rows_v.at[b], gsem.at[b])

        def g_wait(j, b, _y=yc):
            pltpu.make_async_copy(
                _y.at[src_v.at[j]], rows_v.at[b], gsem.at[b]).wait()

        def s_issue(j, b):
            pltpu.async_copy(rows_v.at[b], acc_sh.at[dst_v.at[j]],
                             ssem.at[b], add=True)

        def s_wait(j, b):
            pltpu.make_async_copy(rows_v.at[b], acc_sh.at[dst_v.at[j]],
                                  ssem.at[b]).wait()

        g_issue(0, 0)
        g_issue(1, 1)
        pltpu.sync_copy(zeros_hbm, acc_sh.at[pl.ds(si * ZR, ZR)])
        plsc.subcore_barrier()

        g_wait(0, 0)
        s_issue(0, 0)
        g_wait(1, 1)
        s_issue(1, 1)
        s_wait(0, 0)
        g_issue(2, 0)

        def body(k, carry):
            j0 = 2 * k
            g_wait(j0, 0)
            s_issue(j0, 0)
            s_wait(j0 - 1, 1)
            g_issue(j0 + 1, 1)
            g_wait(j0 + 1, 1)
            s_issue(j0 + 1, 1)
            s_wait(j0, 0)
            g_issue(j0 + 2, 0)
            return carry

        lax.fori_loop(1, NB // 2 - 1, body, 0)

        g_wait(NB - 2, 0)
        s_issue(NB - 2, 0)
        s_wait(NB - 3, 1)
        g_issue(NB - 1, 1)
        g_wait(NB - 1, 1)
        s_issue(NB - 1, 1)
        s_wait(NB - 2, 0)
        s_wait(NB - 1, 1)
        plsc.subcore_barrier()
        pltpu.sync_copy(acc_sh.at[pl.ds(si * ZR, ZR)],
                        outs[c].at[ci].at[pl.ds(si * ZR, ZR)])
        if c + 1 < NCH:
            plsc.subcore_barrier()


@functools.cache
def _scat():
    return pl.kernel(
        _scat_body,
        out_type=[jax.ShapeDtypeStruct((NC, ACC_ROWS, CF), jnp.float32)] * NCH,
        mesh=_mesh(),
        scratch_types=[
            pltpu.VMEM((NB, EB), jnp.int32),
            pltpu.VMEM((NB, EB), jnp.int32),
            pltpu.VMEM((2, EB, CF), jnp.float32),
            pltpu.SemaphoreType.DMA((2,)),
            pltpu.SemaphoreType.DMA((2,)),
            pltpu.VMEM_SHARED((ACC_ROWS, CF), jnp.float32),
        ],
    )


def _mat1_body(x_ref, w_ref, degp_ref, dinv_ref, *y_refs):
    deg = degp_ref[0] + degp_ref[1]
    dinv = lax.rsqrt(deg[:, 0:1] + 1.0)
    dinv_ref[...] = dinv
    xw = jnp.dot(x_ref[...], w_ref[...], preferred_element_type=jnp.float32)
    y = xw * dinv
    for c in range(NCH):
        y_refs[c][...] = y[:, c * CF:(c + 1) * CF]


_mat1 = pl.pallas_call(
    _mat1_body,
    grid=(GRID,),
    in_specs=[
        pl.BlockSpec((RB, DIN), lambda i: (i, 0)),
        pl.BlockSpec((DIN, DH), lambda i: (0, 0)),
        pl.BlockSpec((NC, RB, CF), lambda i: (0, i, 0)),
    ],
    out_specs=[pl.BlockSpec((RB, 1), lambda i: (i, 0))] +
              [pl.BlockSpec((RB, CF), lambda i: (i, 0))] * NCH,
    out_shape=[jax.ShapeDtypeStruct((N, 1), jnp.float32)] +
              [jax.ShapeDtypeStruct((N, CF), jnp.float32)] * NCH,
)


def _mat2_body(dinv_ref, b1_ref, w2_ref,
               a0, a1, a2, a3, y0, y1, y2, y3, *out_refs):
    accs = (a0, a1, a2, a3)
    ys = (y0, y1, y2, y3)
    dinv = dinv_ref[...]
    cols = []
    for c in range(NCH):
        acc = accs[c][0] + accs[c][1] + ys[c][...]
        b = b1_ref[0, c * CF:(c + 1) * CF][None, :]
        cols.append(jnp.maximum(acc * dinv + b, 0.0))
    h1 = jnp.concatenate(cols, axis=1)
    xw = jnp.dot(h1, w2_ref[...], preferred_element_type=jnp.float32)
    ynew = xw * dinv
    for c in range(NCH):
        out_refs[c][...] = ynew[:, c * CF:(c + 1) * CF]


_mat2 = pl.pallas_call(
    _mat2_body,
    grid=(GRID,),
    in_specs=[
        pl.BlockSpec((RB, 1), lambda i: (i, 0)),
        pl.BlockSpec((1, DH), lambda i: (0, 0)),
        pl.BlockSpec((DH, DH), lambda i: (0, 0)),
    ] + [pl.BlockSpec((NC, RB, CF), lambda i: (0, i, 0))] * NCH
      + [pl.BlockSpec((RB, CF), lambda i: (i, 0))] * NCH,
    out_specs=[pl.BlockSpec((RB, CF), lambda i: (i, 0))] * NCH,
    out_shape=[jax.ShapeDtypeStruct((N, CF), jnp.float32)] * NCH,
)


def _mat3_body(dinv_ref, b2_ref, wc1_ref, bc1_ref, wc2_ref, bc2_ref,
               a0, a1, a2, a3, y0, y1, y2, y3,
               h_ref, gr_ref, logits_ref):
    accs = (a0, a1, a2, a3)
    ys = (y0, y1, y2, y3)
    i = pl.program_id(0)
    dinv = dinv_ref[...]
    cols = []
    for c in range(NCH):
        acc = accs[c][0] + accs[c][1] + ys[c][...]
        b = b2_ref[0, c * CF:(c + 1) * CF][None, :]
        cols.append(acc * dinv + b)
    h = jnp.concatenate(cols, axis=1)
    h_ref[...] = h
    partial = jnp.sum(h, axis=0, keepdims=True) * (1.0 / N)

    @pl.when(i == 0)
    def _():
        gr_ref[...] = partial

    @pl.when(i > 0)
    def _():
        gr_ref[...] = gr_ref[...] + partial

    @pl.when(i == GRID - 1)
    def _():
        gr = gr_ref[...]
        cvec = jnp.maximum(
            jnp.dot(gr, wc1_ref[...], preferred_element_type=jnp.float32)
            + bc1_ref[...], 0.0)
        logits_ref[...] = (
            jnp.dot(cvec, wc2_ref[...], preferred_element_type=jnp.float32)
            + bc2_ref[...])


_mat3 = pl.pallas_call(
    _mat3_body,
    grid=(GRID,),
    in_specs=[
        pl.BlockSpec((RB, 1), lambda i: (i, 0)),
        pl.BlockSpec((1, DH), lambda i: (0, 0)),
        pl.BlockSpec((DH, DH), lambda i: (0, 0)),
        pl.BlockSpec((1, DH), lambda i: (0, 0)),
        pl.BlockSpec((DH, DOUT), lambda i: (0, 0)),
        pl.BlockSpec((1, DOUT), lambda i: (0, 0)),
    ] + [pl.BlockSpec((NC, RB, CF), lambda i: (0, i, 0))] * NCH
      + [pl.BlockSpec((RB, CF), lambda i: (i, 0))] * NCH,
    out_specs=[
        pl.BlockSpec((RB, DH), lambda i: (i, 0)),
        pl.BlockSpec((1, DH), lambda i: (0, 0)),
        pl.BlockSpec((1, DOUT), lambda i: (0, 0)),
    ],
    out_shape=[
        jax.ShapeDtypeStruct((N, DH), jnp.float32),
        jax.ShapeDtypeStruct((1, DH), jnp.float32),
        jax.ShapeDtypeStruct((1, DOUT), jnp.float32),
    ],
)


def kernel(x, edge_index, W1, b1, W2, b2, Wc1, bc1, Wc2, bc2):
    src = edge_index[0].astype(jnp.int32)
    dst = edge_index[1].astype(jnp.int32)
    srcp = jnp.pad(src.reshape(NW, EPT),
                   ((0, 0), (0, EPTP - EPT))).reshape(NW, NB, EB)
    dstp = jnp.pad(dst.reshape(NW, EPT), ((0, 0), (0, EPTP - EPT)),
                   constant_values=N).reshape(NW, NB, EB)
    zeros128 = jnp.zeros((ZR, CF), jnp.float32)
    ones128 = jnp.ones((EB, CF), jnp.float32)

    degp = _deg()(dstp, zeros128, ones128)
    dinv, *ycs = _mat1(x, W1, degp)
    acc1 = _scat()(*ycs, srcp, dstp, zeros128)
    y2cs = _mat2(dinv, b1.reshape(1, DH), W2, *acc1, *ycs)
    acc2 = _scat()(*y2cs, srcp, dstp, zeros128)
    h, gr, logits = _mat3(dinv, b2.reshape(1, DH), Wc1, bc1.reshape(1, DH),
                          Wc2, bc2.reshape(1, DOUT), *acc2, *y2cs)
    return h, gr, logits


# speedup vs baseline: 1.0555x; 1.0555x over previous
import functools

import jax
import jax.numpy as jnp
from jax import lax
from jax.experimental import pallas as pl
from jax.experimental.pallas import tpu as pltpu
from jax.experimental.pallas import tpu_sc as plsc

N = 10000
E = 160000
DIN, DH, DOUT = 256, 512, 64
NC, NS = 2, 16
NW = NC * NS
EPT = E // NW
EB = 128
NB = 40
EPTP = NB * EB
CF = 128
NCH = DH // CF
ACC_ROWS = 10112
ZR = ACC_ROWS // NS
RB = 1000
GRID = N // RB

def _mesh():
    return plsc.VectorSubcoreMesh(
        core_axis_name="c", subcore_axis_name="s",
        num_cores=NC, num_subcores=NS)


def _deg_body(dst_hbm, zeros_hbm, ones_hbm, out_hbm, dst_v, ones_v, deg_sh):
    ci = lax.axis_index("c")
    si = lax.axis_index("s")
    wid = ci * NS + si
    pltpu.sync_copy(dst_hbm.at[wid], dst_v)
    pltpu.sync_copy(ones_hbm, ones_v)
    pltpu.sync_copy(zeros_hbm, deg_sh.at[pl.ds(si * ZR, ZR)])
    plsc.subcore_barrier()

    def body(j, carry):
        pltpu.sync_copy(ones_v, deg_sh.at[dst_v.at[j]], add=True)
        return carry

    lax.fori_loop(0, NB, body, 0)
    plsc.subcore_barrier()
    pltpu.sync_copy(deg_sh.at[pl.ds(si * ZR, ZR)],
                    out_hbm.at[ci].at[pl.ds(si * ZR, ZR)])


@functools.cache
def _deg():
    return pl.kernel(
        _deg_body,
        out_type=jax.ShapeDtypeStruct((NC, ACC_ROWS, CF), jnp.float32),
        mesh=_mesh(),
        scratch_types=[
            pltpu.VMEM((NB, EB), jnp.int32),
            pltpu.VMEM((EB, CF), jnp.float32),
            pltpu.VMEM_SHARED((ACC_ROWS, CF), jnp.float32),
        ],
    )


def _scat_body(y0, y1, y2, y3, src_hbm, dst_hbm, zeros_hbm,
               o0, o1, o2, o3, src_v, dst_v, rows_v, gsem, ssem, acc_sh):
    ys = (y0, y1, y2, y3)
    outs = (o0, o1, o2, o3)
    ci = lax.axis_index("c")
    si = lax.axis_index("s")
    wid = ci * NS + si
    pltpu.sync_copy(src_hbm.at[wid], src_v)
    pltpu.sync_copy(dst_hbm.at[wid], dst_v)
    for c in range(NCH):
        yc = ys[c]

        def g_issue(j, b, _y=yc):
            pltpu.async_copy(_y.at[src_v.at[j]], rows_v.at[b], gsem.at[b])

        def g_wait(j, b, _y=yc):
            pltpu.make_async_copy(
                _y.at[src_v.at[j]], rows_v.at[b], gsem.at[b]).wait()

        g_issue(0, 0)
        g_issue(1, 1)
        pltpu.sync_copy(zeros_hbm, acc_sh.at[pl.ds(si * ZR, ZR)])
        plsc.subcore_barrier()

        def body(j0, carry):
            for b in range(2):
                j = 2 * j0 + b
                g_wait(j, b)
                pltpu.sync_copy(rows_v.at[b], acc_sh.at[dst_v.at[j]],
                                add=True)
                g_issue(j + 2, b)
            return carry

        lax.fori_loop(0, NB // 2 - 1, body, 0)
        for b in range(2):
            j = NB - 2 + b
            g_wait(j, b)
            pltpu.sync_copy(rows_v.at[b], acc_sh.at[dst_v.at[j]], add=True)
        plsc.subcore_barrier()
        pltpu.sync_copy(acc_sh.at[pl.ds(si * ZR, ZR)],
                        outs[c].at[ci].at[pl.ds(si * ZR, ZR)])


@functools.cache
def _scat():
    return pl.kernel(
        _scat_body,
        out_type=[jax.ShapeDtypeStruct((NC, ACC_ROWS, CF), jnp.float32)] * NCH,
        mesh=_mesh(),
        scratch_types=[
            pltpu.VMEM((NB, EB), jnp.int32),
            pltpu.VMEM((NB, EB), jnp.int32),
            pltpu.VMEM((2, EB, CF), jnp.float32),
            pltpu.SemaphoreType.DMA((2,)),
            pltpu.SemaphoreType.DMA((2,)),
            pltpu.VMEM_SHARED((ACC_ROWS, CF), jnp.float32),
        ],
    )


def _mat1_body(x_ref, w_ref, degp_ref, dinv_ref, *y_refs):
    deg = degp_ref[0] + degp_ref[1]
    dinv = lax.rsqrt(deg[:, 0:1] + 1.0)
    dinv_ref[...] = dinv
    xw = jnp.dot(x_ref[...], w_ref[...], preferred_element_type=jnp.float32)
    y = xw * dinv
    for c in range(NCH):
        y_refs[c][...] = y[:, c * CF:(c + 1) * CF]


_mat1 = pl.pallas_call(
    _mat1_body,
    grid=(GRID,),
    in_specs=[
        pl.BlockSpec((RB, DIN), lambda i: (i, 0)),
        pl.BlockSpec((DIN, DH), lambda i: (0, 0)),
        pl.BlockSpec((NC, RB, CF), lambda i: (0, i, 0)),
    ],
    out_specs=[pl.BlockSpec((RB, 1), lambda i: (i, 0))] +
              [pl.BlockSpec((RB, CF), lambda i: (i, 0))] * NCH,
    out_shape=[jax.ShapeDtypeStruct((N, 1), jnp.float32)] +
              [jax.ShapeDtypeStruct((N, CF), jnp.float32)] * NCH,
)


def _mat2_body(dinv_ref, b1_ref, w2_ref,
               a0, a1, a2, a3, y0, y1, y2, y3, *out_refs):
    accs = (a0, a1, a2, a3)
    ys = (y0, y1, y2, y3)
    dinv = dinv_ref[...]
    cols = []
    for c in range(NCH):
        acc = accs[c][0] + accs[c][1] + ys[c][...]
        b = b1_ref[0, c * CF:(c + 1) * CF][None, :]
        cols.append(jnp.maximum(acc * dinv + b, 0.0))
    h1 = jnp.concatenate(cols, axis=1)
    xw = jnp.dot(h1, w2_ref[...], preferred_element_type=jnp.float32)
    ynew = xw * dinv
    for c in range(NCH):
        out_refs[c][...] = ynew[:, c * CF:(c + 1) * CF]


_mat2 = pl.pallas_call(
    _mat2_body,
    grid=(GRID,),
    in_specs=[
        pl.BlockSpec((RB, 1), lambda i: (i, 0)),
        pl.BlockSpec((1, DH), lambda i: (0, 0)),
        pl.BlockSpec((DH, DH), lambda i: (0, 0)),
    ] + [pl.BlockSpec((NC, RB, CF), lambda i: (0, i, 0))] * NCH
      + [pl.BlockSpec((RB, CF), lambda i: (i, 0))] * NCH,
    out_specs=[pl.BlockSpec((RB, CF), lambda i: (i, 0))] * NCH,
    out_shape=[jax.ShapeDtypeStruct((N, CF), jnp.float32)] * NCH,
)


def _mat3_body(dinv_ref, b2_ref, wc1_ref, bc1_ref, wc2_ref, bc2_ref,
               a0, a1, a2, a3, y0, y1, y2, y3,
               h_ref, gr_ref, logits_ref):
    accs = (a0, a1, a2, a3)
    ys = (y0, y1, y2, y3)
    i = pl.program_id(0)
    dinv = dinv_ref[...]
    cols = []
    for c in range(NCH):
        acc = accs[c][0] + accs[c][1] + ys[c][...]
        b = b2_ref[0, c * CF:(c + 1) * CF][None, :]
        cols.append(acc * dinv + b)
    h = jnp.concatenate(cols, axis=1)
    h_ref[...] = h
    partial = jnp.sum(h, axis=0, keepdims=True) * (1.0 / N)

    @pl.when(i == 0)
    def _():
        gr_ref[...] = partial

    @pl.when(i > 0)
    def _():
        gr_ref[...] = gr_ref[...] + partial

    @pl.when(i == GRID - 1)
    def _():
        gr = gr_ref[...]
        cvec = jnp.maximum(
            jnp.dot(gr, wc1_ref[...], preferred_element_type=jnp.float32)
            + bc1_ref[...], 0.0)
        logits_ref[...] = (
            jnp.dot(cvec, wc2_ref[...], preferred_element_type=jnp.float32)
            + bc2_ref[...])


_mat3 = pl.pallas_call(
    _mat3_body,
    grid=(GRID,),
    in_specs=[
        pl.BlockSpec((RB, 1), lambda i: (i, 0)),
        pl.BlockSpec((1, DH), lambda i: (0, 0)),
        pl.BlockSpec((DH, DH), lambda i: (0, 0)),
        pl.BlockSpec((1, DH), lambda i: (0, 0)),
        pl.BlockSpec((DH, DOUT), lambda i: (0, 0)),
        pl.BlockSpec((1, DOUT), lambda i: (0, 0)),
    ] + [pl.BlockSpec((NC, RB, CF), lambda i: (0, i, 0))] * NCH
      + [pl.BlockSpec((RB, CF), lambda i: (i, 0))] * NCH,
    out_specs=[
        pl.BlockSpec((RB, DH), lambda i: (i, 0)),
        pl.BlockSpec((1, DH), lambda i: (0, 0)),
        pl.BlockSpec((1, DOUT), lambda i: (0, 0)),
    ],
    out_shape=[
        jax.ShapeDtypeStruct((N, DH), jnp.float32),
        jax.ShapeDtypeStruct((1, DH), jnp.float32),
        jax.ShapeDtypeStruct((1, DOUT), jnp.float32),
    ],
)


def kernel(x, edge_index, W1, b1, W2, b2, Wc1, bc1, Wc2, bc2):
    src = edge_index[0].astype(jnp.int32)
    dst = edge_index[1].astype(jnp.int32)
    srcp = jnp.pad(src.reshape(NW, EPT),
                   ((0, 0), (0, EPTP - EPT))).reshape(NW, NB, EB)
    dstp = jnp.pad(dst.reshape(NW, EPT), ((0, 0), (0, EPTP - EPT)),
                   constant_values=N).reshape(NW, NB, EB)
    zeros128 = jnp.zeros((ZR, CF), jnp.float32)
    ones128 = jnp.ones((EB, CF), jnp.float32)

    degp = _deg()(dstp, zeros128, ones128)
    dinv, *ycs = _mat1(x, W1, degp)
    acc1 = _scat()(*ycs, srcp, dstp, zeros128)
    y2cs = _mat2(dinv, b1.reshape(1, DH), W2, *acc1, *ycs)
    acc2 = _scat()(*y2cs, srcp, dstp, zeros128)
    h, gr, logits = _mat3(dinv, b2.reshape(1, DH), Wc1, bc1.reshape(1, DH),
                          Wc2, bc2.reshape(1, DOUT), *acc2, *y2cs)
    return h, gr, logits


# speedup vs baseline: 1.0741x; 1.0176x over previous
import functools

import jax
import jax.numpy as jnp
from jax import lax
from jax.experimental import pallas as pl
from jax.experimental.pallas import tpu as pltpu
from jax.experimental.pallas import tpu_sc as plsc

N = 10000
E = 160000
DIN, DH, DOUT = 256, 512, 64
NC, NS = 2, 16
NW = NC * NS
EPT = E // NW
EB = 128
NB = 40
EPTP = NB * EB
CF = 128
NCH = DH // CF
ACC_ROWS = 10112
ZR = ACC_ROWS // NS
RB = 1000
GRID = N // RB

def _mesh():
    return plsc.VectorSubcoreMesh(
        core_axis_name="c", subcore_axis_name="s",
        num_cores=NC, num_subcores=NS)


def _deg_body(dst_hbm, zeros_hbm, ones_hbm, out_hbm, dst_v, ones_v, deg_sh):
    ci = lax.axis_index("c")
    si = lax.axis_index("s")
    wid = ci * NS + si
    pltpu.sync_copy(dst_hbm.at[wid], dst_v)
    pltpu.sync_copy(ones_hbm, ones_v)
    pltpu.sync_copy(zeros_hbm, deg_sh.at[pl.ds(si * ZR, ZR)])
    plsc.subcore_barrier()

    def body(j, carry):
        pltpu.sync_copy(ones_v, deg_sh.at[dst_v.at[j]], add=True)
        return carry

    lax.fori_loop(0, NB, body, 0)
    plsc.subcore_barrier()
    pltpu.sync_copy(deg_sh.at[pl.ds(si * ZR, ZR)],
                    out_hbm.at[ci].at[pl.ds(si * ZR, ZR)])


@functools.cache
def _deg():
    return pl.kernel(
        _deg_body,
        out_type=jax.ShapeDtypeStruct((NC, ACC_ROWS, CF), jnp.float32),
        mesh=_mesh(),
        scratch_types=[
            pltpu.VMEM((NB, EB), jnp.int32),
            pltpu.VMEM((EB, CF), jnp.float32),
            pltpu.VMEM_SHARED((ACC_ROWS, CF), jnp.float32),
        ],
    )


def _scat_body(y0, y1, y2, y3, src_hbm, dst_hbm, zeros_hbm,
               o0, o1, o2, o3, src_v, dst_v, rows_v, gsem, ssem, acc_sh):
    ys = (y0, y1, y2, y3)
    outs = (o0, o1, o2, o3)
    ci = lax.axis_index("c")
    si = lax.axis_index("s")
    wid = ci * NS + si
    pltpu.sync_copy(src_hbm.at[wid], src_v)
    pltpu.sync_copy(dst_hbm.at[wid], dst_v)
    for c in range(NCH):
        yc = ys[c]

        def g_issue(j, b, _y=yc):
            pltpu.async_copy(_y.at[src_v.at[j]], rows_v.at[b], gsem.at[b])

        def g_wait(j, b, _y=yc):
            pltpu.make_async_copy(
                _y.at[src_v.at[j]], rows_v.at[b], gsem.at[b]).wait()

        g_issue(0, 0)
        g_issue(1, 1)
        if c == 0:
            pltpu.sync_copy(zeros_hbm, acc_sh.at[pl.ds(si * ZR, ZR)])
        plsc.subcore_barrier()

        def body(j0, carry):
            for b in range(2):
                j = 2 * j0 + b
                g_wait(j, b)
                pltpu.sync_copy(rows_v.at[b], acc_sh.at[dst_v.at[j]],
                                add=True)
                g_issue(j + 2, b)
            return carry

        lax.fori_loop(0, NB // 2 - 1, body, 0)
        for b in range(2):
            j = NB - 2 + b
            g_wait(j, b)
            pltpu.sync_copy(rows_v.at[b], acc_sh.at[dst_v.at[j]], add=True)
        plsc.subcore_barrier()
        pltpu.sync_copy(acc_sh.at[pl.ds(si * ZR, ZR)],
                        outs[c].at[ci].at[pl.ds(si * ZR, ZR)])


@functools.cache
def _scat():
    return pl.kernel(
        _scat_body,
        out_type=[jax.ShapeDtypeStruct((NC, ACC_ROWS, CF), jnp.float32)] * NCH,
        mesh=_mesh(),
        scratch_types=[
            pltpu.VMEM((NB, EB), jnp.int32),
            pltpu.VMEM((NB, EB), jnp.int32),
            pltpu.VMEM((2, EB, CF), jnp.float32),
            pltpu.SemaphoreType.DMA((2,)),
            pltpu.SemaphoreType.DMA((2,)),
            pltpu.VMEM_SHARED((ACC_ROWS, CF), jnp.float32),
        ],
    )


def _mat1_body(x_ref, w_ref, degp_ref, dinv_ref, *y_refs):
    deg = degp_ref[0] + degp_ref[1]
    dinv = lax.rsqrt(deg[:, 0:1] + 1.0)
    dinv_ref[...] = dinv
    xw = jnp.dot(x_ref[...], w_ref[...], preferred_element_type=jnp.float32)
    y = xw * dinv
    for c in range(NCH):
        y_refs[c][...] = y[:, c * CF:(c + 1) * CF]


_mat1 = pl.pallas_call(
    _mat1_body,
    grid=(GRID,),
    in_specs=[
        pl.BlockSpec((RB, DIN), lambda i: (i, 0)),
        pl.BlockSpec((DIN, DH), lambda i: (0, 0)),
        pl.BlockSpec((NC, RB, CF), lambda i: (0, i, 0)),
    ],
    out_specs=[pl.BlockSpec((RB, 1), lambda i: (i, 0))] +
              [pl.BlockSpec((RB, CF), lambda i: (i, 0))] * NCH,
    out_shape=[jax.ShapeDtypeStruct((N, 1), jnp.float32)] +
              [jax.ShapeDtypeStruct((N, CF), jnp.float32)] * NCH,
)


def _mat2_body(dinv_ref, b1_ref, w2_ref,
               a0, a1, a2, a3, y0, y1, y2, y3, *out_refs):
    accs = (a0, a1, a2, a3)
    ys = (y0, y1, y2, y3)
    dinv = dinv_ref[...]
    cols = []
    prev = None
    for c in range(NCH):
        cum = accs[c][0] + accs[c][1]
        acc = (cum if prev is None else cum - prev) + ys[c][...]
        prev = cum
        b = b1_ref[0, c * CF:(c + 1) * CF][None, :]
        cols.append(jnp.maximum(acc * dinv + b, 0.0))
    h1 = jnp.concatenate(cols, axis=1)
    xw = jnp.dot(h1, w2_ref[...], preferred_element_type=jnp.float32)
    ynew = xw * dinv
    for c in range(NCH):
        out_refs[c][...] = ynew[:, c * CF:(c + 1) * CF]


_mat2 = pl.pallas_call(
    _mat2_body,
    grid=(GRID,),
    in_specs=[
        pl.BlockSpec((RB, 1), lambda i: (i, 0)),
        pl.BlockSpec((1, DH), lambda i: (0, 0)),
        pl.BlockSpec((DH, DH), lambda i: (0, 0)),
    ] + [pl.BlockSpec((NC, RB, CF), lambda i: (0, i, 0))] * NCH
      + [pl.BlockSpec((RB, CF), lambda i: (i, 0))] * NCH,
    out_specs=[pl.BlockSpec((RB, CF), lambda i: (i, 0))] * NCH,
    out_shape=[jax.ShapeDtypeStruct((N, CF), jnp.float32)] * NCH,
)


def _mat3_body(dinv_ref, b2_ref, wc1_ref, bc1_ref, wc2_ref, bc2_ref,
               a0, a1, a2, a3, y0, y1, y2, y3,
               h_ref, gr_ref, logits_ref):
    accs = (a0, a1, a2, a3)
    ys = (y0, y1, y2, y3)
    i = pl.program_id(0)
    dinv = dinv_ref[...]
    cols = []
    prev = None
    for c in range(NCH):
        cum = accs[c][0] + accs[c][1]
        acc = (cum if prev is None else cum - prev) + ys[c][...]
        prev = cum
        b = b2_ref[0, c * CF:(c + 1) * CF][None, :]
        cols.append(acc * dinv + b)
    h = jnp.concatenate(cols, axis=1)
    h_ref[...] = h
    partial = jnp.sum(h, axis=0, keepdims=True) * (1.0 / N)

    @pl.when(i == 0)
    def _():
        gr_ref[...] = partial

    @pl.when(i > 0)
    def _():
        gr_ref[...] = gr_ref[...] + partial

    @pl.when(i == GRID - 1)
    def _():
        gr = gr_ref[...]
        cvec = jnp.maximum(
            jnp.dot(gr, wc1_ref[...], preferred_element_type=jnp.float32)
            + bc1_ref[...], 0.0)
        logits_ref[...] = (
            jnp.dot(cvec, wc2_ref[...], preferred_element_type=jnp.float32)
            + bc2_ref[...])


_mat3 = pl.pallas_call(
    _mat3_body,
    grid=(GRID,),
    in_specs=[
        pl.BlockSpec((RB, 1), lambda i: (i, 0)),
        pl.BlockSpec((1, DH), lambda i: (0, 0)),
        pl.BlockSpec((DH, DH), lambda i: (0, 0)),
        pl.BlockSpec((1, DH), lambda i: (0, 0)),
        pl.BlockSpec((DH, DOUT), lambda i: (0, 0)),
        pl.BlockSpec((1, DOUT), lambda i: (0, 0)),
    ] + [pl.BlockSpec((NC, RB, CF), lambda i: (0, i, 0))] * NCH
      + [pl.BlockSpec((RB, CF), lambda i: (i, 0))] * NCH,
    out_specs=[
        pl.BlockSpec((RB, DH), lambda i: (i, 0)),
        pl.BlockSpec((1, DH), lambda i: (0, 0)),
        pl.BlockSpec((1, DOUT), lambda i: (0, 0)),
    ],
    out_shape=[
        jax.ShapeDtypeStruct((N, DH), jnp.float32),
        jax.ShapeDtypeStruct((1, DH), jnp.float32),
        jax.ShapeDtypeStruct((1, DOUT), jnp.float32),
    ],
)


def kernel(x, edge_index, W1, b1, W2, b2, Wc1, bc1, Wc2, bc2):
    src = edge_index[0].astype(jnp.int32)
    dst = edge_index[1].astype(jnp.int32)
    srcp = jnp.pad(src.reshape(NW, EPT),
                   ((0, 0), (0, EPTP - EPT))).reshape(NW, NB, EB)
    dstp = jnp.pad(dst.reshape(NW, EPT), ((0, 0), (0, EPTP - EPT)),
                   constant_values=N).reshape(NW, NB, EB)
    zeros128 = jnp.zeros((ZR, CF), jnp.float32)
    ones128 = jnp.ones((EB, CF), jnp.float32)

    degp = _deg()(dstp, zeros128, ones128)
    dinv, *ycs = _mat1(x, W1, degp)
    acc1 = _scat()(*ycs, srcp, dstp, zeros128)
    y2cs = _mat2(dinv, b1.reshape(1, DH), W2, *acc1, *ycs)
    acc2 = _scat()(*y2cs, srcp, dstp, zeros128)
    h, gr, logits = _mat3(dinv, b2.reshape(1, DH), Wc1, bc1.reshape(1, DH),
                          Wc2, bc2.reshape(1, DOUT), *acc2, *y2cs)
    return h, gr, logits
